# baseline (speedup 1.0000x reference)
import functools

import jax
import jax.numpy as jnp
from jax.experimental import pallas as pl

_KFRAC = 0.2
_ALPHA = 0.7
_BITS = 26


def _pool_body(x_ref, o_ref, *, k_top, n):
    x = x_ref[...]
    rows = x.shape[0]
    k_bot = n - k_top + 1

    mx = jnp.max(x, axis=1, keepdims=True)
    mn = jnp.min(x, axis=1, keepdims=True)

    ones = jnp.ones((n, 1), jnp.float32)

    def rowsum(v):
        return jax.lax.dot(v, ones, precision=jax.lax.Precision.HIGHEST)

    def step(_, carry):
        lo_a, hi_a, lo_b, hi_b = carry
        mid_a = 0.5 * (lo_a + hi_a)
        mid_b = 0.5 * (lo_b + hi_b)
        c_a = rowsum(jnp.where(x >= mid_a, 1.0, 0.0))
        c_b = rowsum(jnp.where(x >= mid_b, 1.0, 0.0))
        ok_a = c_a >= k_top
        ok_b = c_b >= k_bot
        lo_a = jnp.where(ok_a, mid_a, lo_a)
        hi_a = jnp.where(ok_a, hi_a, mid_a)
        lo_b = jnp.where(ok_b, mid_b, lo_b)
        hi_b = jnp.where(ok_b, hi_b, mid_b)
        return lo_a, hi_a, lo_b, hi_b

    lo_a, _, lo_b, _ = jax.lax.fori_loop(0, _BITS, step, (mn, mx, mn, mx))

    s_top = k_top * lo_a[:, 0] + rowsum(jnp.maximum(x - lo_a, 0.0))[:, 0]
    s_bot = k_top * lo_b[:, 0] - rowsum(jnp.maximum(lo_b - x, 0.0))[:, 0]
    out = (s_top + _ALPHA * s_bot) * (0.5 / k_top)
    o_ref[...] = out.reshape(1, 1, rows)


def kernel(input):
    b, c, h, w = input.shape
    n = h * w
    k_top = int(round(_KFRAC * n))
    rows = b * c
    r_blk = 256
    grid = rows // r_blk
    x = input.reshape(rows, n)

    out = pl.pallas_call(
        functools.partial(_pool_body, k_top=k_top, n=n),
        grid=(grid,),
        in_specs=[pl.BlockSpec((r_blk, n), lambda i: (i, 0))],
        out_specs=pl.BlockSpec((1, 1, r_blk), lambda i: (i, 0, 0)),
        out_shape=jax.ShapeDtypeStruct((grid, 1, r_blk), jnp.float32),
    )(x)
    return out.reshape(b, c)


# speedup vs baseline: 5.7276x; 5.7276x over previous
import functools

import jax
import jax.numpy as jnp
from jax.experimental import pallas as pl

_KFRAC = 0.2
_ALPHA = 0.7
_BITS = 20


def _pool_body(x_ref, o_ref, *, k_top, n):
    x = x_ref[...]
    rows = x.shape[0]
    k_bot = n - k_top + 1

    mx = jnp.max(x, axis=1, keepdims=True)
    mn = jnp.min(x, axis=1, keepdims=True)

    def step(_, carry):
        lo_a, hi_a, lo_b, hi_b = carry
        mid_a = 0.5 * (lo_a + hi_a)
        mid_b = 0.5 * (lo_b + hi_b)
        comb = jnp.where(x >= mid_a, jnp.int32(1), jnp.int32(0)) + jnp.where(
            x >= mid_b, jnp.int32(2048), jnp.int32(0)
        )
        cnt = jnp.sum(comb, axis=1, keepdims=True)
        c_a = cnt & jnp.int32(2047)
        c_b = jax.lax.shift_right_logical(cnt, jnp.int32(11))
        ok_a = c_a >= k_top
        ok_b = c_b >= k_bot
        lo_a = jnp.where(ok_a, mid_a, lo_a)
        hi_a = jnp.where(ok_a, hi_a, mid_a)
        lo_b = jnp.where(ok_b, mid_b, lo_b)
        hi_b = jnp.where(ok_b, hi_b, mid_b)
        return lo_a, hi_a, lo_b, hi_b

    lo_a, _, lo_b, _ = jax.lax.fori_loop(0, _BITS, step, (mn, mx, mn, mx), unroll=4)

    s_top = k_top * lo_a[:, 0] + jnp.sum(jnp.maximum(x - lo_a, 0.0), axis=1)
    s_bot = k_top * lo_b[:, 0] - jnp.sum(jnp.maximum(lo_b - x, 0.0), axis=1)
    out = (s_top + _ALPHA * s_bot) * (0.5 / k_top)
    o_ref[...] = out.reshape(1, 1, rows)


def kernel(input):
    b, c, h, w = input.shape
    n = h * w
    k_top = int(round(_KFRAC * n))
    rows = b * c
    r_blk = 256
    grid = rows // r_blk
    x = input.reshape(rows, n)

    out = pl.pallas_call(
        functools.partial(_pool_body, k_top=k_top, n=n),
        grid=(grid,),
        in_specs=[pl.BlockSpec((r_blk, n), lambda i: (i, 0))],
        out_specs=pl.BlockSpec((1, 1, r_blk), lambda i: (i, 0, 0)),
        out_shape=jax.ShapeDtypeStruct((grid, 1, r_blk), jnp.float32),
    )(x)
    return out.reshape(b, c)
